# baseline (speedup 1.0000x reference)
import jax
import jax.numpy as jnp
from jax import lax
from jax.experimental import pallas as pl
from jax.experimental.pallas import tpu as pltpu
from jax.experimental.pallas import tpu_sc as plsc

_N = 10000
_E = 160000
_NP = 10752
_HID = 512
_HALF = 256
_BM = 128
_NBLK = _NP // _BM
_RPT = _NP // 32
_AW = _RPT * _HALF
_K = 48
_STRIP = 3200
_NSTRIP = _E // _STRIP
_EPT32 = _E // 32
_CROWS = 768

_mesh = plsc.VectorSubcoreMesh(core_axis_name="c", subcore_axis_name="s")
_params = pltpu.CompilerParams(needs_layout_passes=False)


def _count_body(dst_hbm, out_hbm, dbuf, cnt, sem):
    c = lax.axis_index("c")
    s = lax.axis_index("s")
    w = c * 16 + s
    pltpu.async_copy(dst_hbm.at[pl.ds(w * _EPT32, _EPT32)],
                     dbuf.at[pl.ds(0, _EPT32)], sem).wait()
    zf = jnp.zeros((16,), jnp.float32)
    for r in range(_CROWS):
        cnt[pl.ds(r * 16, 16)] = zf
    ones = jnp.ones((16,), jnp.float32)
    iota = lax.iota(jnp.int32, 16)

    def body(i, _):
        off = i * 16
        v = dbuf[pl.ds(off, 16)]
        m = (off + iota) < _EPT32
        plsc.addupdate_scatter(cnt, [v], ones, mask=m)
        return 0

    lax.fori_loop(0, (_EPT32 + 15) // 16, body, 0)
    pltpu.sync_copy(cnt, out_hbm.at[w])


def _count(dst):
    f = pl.kernel(
        _count_body,
        out_type=jax.ShapeDtypeStruct((32, _CROWS * 16), jnp.float32),
        mesh=_mesh,
        compiler_params=_params,
        scratch_types=[
            pltpu.VMEM((_EPT32 + 8,), jnp.int32),
            pltpu.VMEM((_CROWS * 16,), jnp.float32),
            pltpu.SemaphoreType.DMA,
        ],
    )
    return f(dst)


def _agg_body(glo_hbm, ghi_hbm, src_hbm, dst_hbm, slo_hbm, shi_hbm,
              sbuf0, sbuf1, dbuf0, dbuf1, csrc, cdst, gbuf0, gbuf1, acc,
              seme0, seme1, semg0, semg1, sem):
    c = lax.axis_index("c")
    s = lax.axis_index("s")
    w = c * 16 + s
    lo = w * _RPT
    iota = lax.iota(jnp.int32, 16)
    zf = jnp.zeros((16,), jnp.float32)
    dummy_s = jnp.full((16,), _NP - 1, jnp.int32)
    dummy_d = jnp.zeros((16,), jnp.int32)
    cols = [j * 16 + iota for j in range(_HALF // 16)]
    lane = [jnp.full((16,), k, jnp.int32) for k in range(16)]
    sb = (sbuf0, sbuf1)
    db = (dbuf0, dbuf1)
    gb = (gbuf0, gbuf1)
    seme = (seme0, seme1)
    semg = (semg0, semg1)

    for half in range(2):
        g_hbm = glo_hbm if half == 0 else ghi_hbm
        o_hbm = slo_hbm if half == 0 else shi_hbm

        def zero(i, _):
            acc[pl.ds(i * 16, 16)] = zf
            return 0

        lax.fori_loop(0, _AW // 16, zero, 0)

        for b in range(2):
            pltpu.async_copy(src_hbm.at[pl.ds(b * _STRIP, _STRIP)],
                             sb[b], seme[b])
            pltpu.async_copy(dst_hbm.at[pl.ds(b * _STRIP, _STRIP)],
                             db[b], seme[b])

        def spair(sg, _s):
            for b in range(2):
                i = sg * 2 + b
                base_e = i * _STRIP
                sbuf, dbuf, sem_e = sb[b], db[b], seme[b]
                pltpu.make_async_copy(
                    src_hbm.at[pl.ds(base_e, _STRIP)], sbuf, sem_e).wait()
                pltpu.make_async_copy(
                    dst_hbm.at[pl.ds(base_e, _STRIP)], dbuf, sem_e).wait()

                def scan(ii, n):
                    dv = dbuf[pl.ds(ii * 16, 16)]
                    sv = sbuf[pl.ds(ii * 16, 16)]
                    m = (dv >= lo) & (dv < lo + _RPT)
                    plsc.store_compressed(csrc.at[pl.ds(n, 16)], sv, mask=m)
                    plsc.store_compressed(cdst.at[pl.ds(n, 16)], dv - lo,
                                          mask=m)
                    return n + jnp.sum(m.astype(jnp.int32))

                cnt = lax.fori_loop(0, _STRIP // 16, scan, jnp.int32(0))
                for k in range(_K // 16 + 1):
                    csrc[pl.ds(cnt + k * 16, 16)] = dummy_s
                    cdst[pl.ds(cnt + k * 16, 16)] = dummy_d

                @pl.when(i + 2 < _NSTRIP)
                def _():
                    nb = (i + 2) * _STRIP
                    pltpu.async_copy(src_hbm.at[pl.ds(nb, _STRIP)], sbuf,
                                     sem_e)
                    pltpu.async_copy(dst_hbm.at[pl.ds(nb, _STRIP)], dbuf,
                                     sem_e)

                nch = (cnt + _K - 1) // _K

                @pl.when(nch > 0)
                def _():
                    pltpu.async_copy(g_hbm.at[csrc.at[pl.ds(0, _K)]],
                                     gb[0], semg[0])

                def cpair(cg, _c):
                    for b2 in range(2):
                        cidx = cg * 2 + b2
                        gbuf, sem_g = gb[b2], semg[b2]

                        @pl.when(cidx < nch)
                        def _():
                            cb = cidx * _K
                            pltpu.make_async_copy(
                                g_hbm.at[csrc.at[pl.ds(cb, _K)]], gbuf,
                                sem_g).wait()

                            @pl.when(cidx + 1 < nch)
                            def _():
                                nxt = (cidx + 1) * _K
                                pltpu.async_copy(
                                    g_hbm.at[csrc.at[pl.ds(nxt, _K)]],
                                    gb[1 - b2], semg[1 - b2])

                            def edges16(q, _2):
                                dv16 = cdst[pl.ds(cb + q * 16, 16)]
                                rowb16 = dv16 * _HALF
                                for k in range(16):
                                    rowb = rowb16[lane[k]]
                                    gr = q * 16 + k
                                    for j in range(_HALF // 16):
                                        val = gbuf[gr, j // 8,
                                                   pl.ds((j % 8) * 16, 16)]
                                        plsc.addupdate_scatter(
                                            acc, [rowb + cols[j]], val)
                                return 0

                            lax.fori_loop(0, _K // 16, edges16, 0)

                    return 0

                lax.fori_loop(0, (nch + 1) // 2, cpair, 0)
            return 0

        lax.fori_loop(0, _NSTRIP // 2, spair, 0)
        pltpu.sync_copy(acc, o_hbm.at[pl.ds(w * _AW, _AW)])


def _agg(glo, ghi, src, dst):
    f = pl.kernel(
        _agg_body,
        out_type=[
            jax.ShapeDtypeStruct((_NP * _HALF,), jnp.float32),
            jax.ShapeDtypeStruct((_NP * _HALF,), jnp.float32),
        ],
        mesh=_mesh,
        compiler_params=_params,
        scratch_types=[
            pltpu.VMEM((_STRIP,), jnp.int32),
            pltpu.VMEM((_STRIP,), jnp.int32),
            pltpu.VMEM((_STRIP,), jnp.int32),
            pltpu.VMEM((_STRIP,), jnp.int32),
            pltpu.VMEM((_STRIP + 2 * _K,), jnp.int32),
            pltpu.VMEM((_STRIP + 2 * _K,), jnp.int32),
            pltpu.VMEM((_K, _HALF // 128, 128), jnp.float32),
            pltpu.VMEM((_K, _HALF // 128, 128), jnp.float32),
            pltpu.VMEM((_AW,), jnp.float32),
            pltpu.SemaphoreType.DMA,
            pltpu.SemaphoreType.DMA,
            pltpu.SemaphoreType.DMA,
            pltpu.SemaphoreType.DMA,
            pltpu.SemaphoreType.DMA,
        ],
    )
    return f(glo.reshape(_NP, _HALF // 128, 128),
             ghi.reshape(_NP, _HALF // 128, 128), src, dst)


def _mm1_body(x_ref, w_ref, p_ref, glo_ref, ghi_ref, dis_ref):
    j = pl.program_id(0)
    deg = jnp.sum(p_ref[...], axis=1, keepdims=True) + 1.0
    rid = j * _BM + lax.broadcasted_iota(jnp.int32, (_BM, 1), 0)
    dis = jnp.where(rid < _N, lax.rsqrt(deg), 0.0)
    g = jnp.dot(x_ref[...], w_ref[...],
                preferred_element_type=jnp.float32) * dis
    glo_ref[...] = g[:, :_HALF]
    ghi_ref[...] = g[:, _HALF:]
    dis_ref[...] = dis


def _mm1(x_p, W1, parts):
    return pl.pallas_call(
        _mm1_body,
        grid=(_NBLK,),
        in_specs=[
            pl.BlockSpec((_BM, 256), lambda j: (j, 0)),
            pl.BlockSpec((256, _HID), lambda j: (0, 0)),
            pl.BlockSpec((_BM, 32), lambda j: (j, 0)),
        ],
        out_specs=[
            pl.BlockSpec((_BM, _HALF), lambda j: (j, 0)),
            pl.BlockSpec((_BM, _HALF), lambda j: (j, 0)),
            pl.BlockSpec((_BM, 1), lambda j: (j, 0)),
        ],
        out_shape=[
            jax.ShapeDtypeStruct((_NP, _HALF), jnp.float32),
            jax.ShapeDtypeStruct((_NP, _HALF), jnp.float32),
            jax.ShapeDtypeStruct((_NP, 1), jnp.float32),
        ],
    )(x_p, W1, parts)


def _mm2_body(slo_ref, shi_ref, glo_ref, ghi_ref, dis_ref, b_ref, w_ref,
              olo_ref, ohi_ref):
    dis = dis_ref[...]
    s = jnp.concatenate([slo_ref[...], shi_ref[...]], axis=1)
    g = jnp.concatenate([glo_ref[...], ghi_ref[...]], axis=1)
    h = jnp.maximum(dis * (s + g) + b_ref[...], 0.0)
    o = jnp.dot(h, w_ref[...], preferred_element_type=jnp.float32) * dis
    olo_ref[...] = o[:, :_HALF]
    ohi_ref[...] = o[:, _HALF:]


def _mm2(s1lo, s1hi, g1lo, g1hi, dis, b1, W2):
    return pl.pallas_call(
        _mm2_body,
        grid=(_NBLK,),
        in_specs=[
            pl.BlockSpec((_BM, _HALF), lambda j: (j, 0)),
            pl.BlockSpec((_BM, _HALF), lambda j: (j, 0)),
            pl.BlockSpec((_BM, _HALF), lambda j: (j, 0)),
            pl.BlockSpec((_BM, _HALF), lambda j: (j, 0)),
            pl.BlockSpec((_BM, 1), lambda j: (j, 0)),
            pl.BlockSpec((1, _HID), lambda j: (0, 0)),
            pl.BlockSpec((_HID, _HID), lambda j: (0, 0)),
        ],
        out_specs=[
            pl.BlockSpec((_BM, _HALF), lambda j: (j, 0)),
            pl.BlockSpec((_BM, _HALF), lambda j: (j, 0)),
        ],
        out_shape=[
            jax.ShapeDtypeStruct((_NP, _HALF), jnp.float32),
            jax.ShapeDtypeStruct((_NP, _HALF), jnp.float32),
        ],
    )(s1lo, s1hi, g1lo, g1hi, dis, b1, W2)


def _mm3_body(slo_ref, shi_ref, glo_ref, ghi_ref, dis_ref, b_ref, wc_ref,
              bc_ref, out_ref):
    dis = dis_ref[...]
    s = jnp.concatenate([slo_ref[...], shi_ref[...]], axis=1)
    g = jnp.concatenate([glo_ref[...], ghi_ref[...]], axis=1)
    h = jnp.maximum(dis * (s + g) + b_ref[...], 0.0)
    out_ref[...] = jnp.dot(h, wc_ref[...],
                           preferred_element_type=jnp.float32) + bc_ref[...]


def _mm3(s2lo, s2hi, g2lo, g2hi, dis, b2, Wcp, bcp):
    return pl.pallas_call(
        _mm3_body,
        grid=(_NBLK,),
        in_specs=[
            pl.BlockSpec((_BM, _HALF), lambda j: (j, 0)),
            pl.BlockSpec((_BM, _HALF), lambda j: (j, 0)),
            pl.BlockSpec((_BM, _HALF), lambda j: (j, 0)),
            pl.BlockSpec((_BM, _HALF), lambda j: (j, 0)),
            pl.BlockSpec((_BM, 1), lambda j: (j, 0)),
            pl.BlockSpec((1, _HID), lambda j: (0, 0)),
            pl.BlockSpec((_HID, _BM), lambda j: (0, 0)),
            pl.BlockSpec((1, _BM), lambda j: (0, 0)),
        ],
        out_specs=pl.BlockSpec((_BM, _BM), lambda j: (j, 0)),
        out_shape=jax.ShapeDtypeStruct((_NP, _BM), jnp.float32),
    )(s2lo, s2hi, g2lo, g2hi, dis, b2, Wcp, bcp)


def kernel(x, edge_index, W1, b1, W2, b2, Wc, bc):
    src = edge_index[0]
    dst = edge_index[1]
    x_p = jnp.pad(x, ((0, _NP - _N), (0, 0)))

    counts = _count(dst)
    parts = counts[:, :_NP].T

    g1lo, g1hi, dis = _mm1(x_p, W1, parts)
    s1lo, s1hi = _agg(g1lo, g1hi, src, dst)
    s1lo = s1lo.reshape(_NP, _HALF)
    s1hi = s1hi.reshape(_NP, _HALF)
    g2lo, g2hi = _mm2(s1lo, s1hi, g1lo, g1hi, dis, b1.reshape(1, -1), W2)
    s2lo, s2hi = _agg(g2lo, g2hi, src, dst)
    s2lo = s2lo.reshape(_NP, _HALF)
    s2hi = s2hi.reshape(_NP, _HALF)
    Wcp = jnp.pad(Wc, ((0, 0), (0, _BM - Wc.shape[1])))
    bcp = jnp.pad(bc, (0, _BM - bc.shape[0])).reshape(1, -1)
    out = _mm3(s2lo, s2hi, g2lo, g2hi, dis, b2.reshape(1, -1), Wcp, bcp)
    return out[:_N, :Wc.shape[1]]


# speedup vs baseline: 2.9195x; 2.9195x over previous
import jax
import jax.numpy as jnp
from jax import lax
from jax.experimental import pallas as pl
from jax.experimental.pallas import tpu as pltpu
from jax.experimental.pallas import tpu_sc as plsc

_N = 10000
_E = 160000
_NP = 10752
_HID = 512
_HALF = 256
_BM = 128
_NBLK = _NP // _BM
_RPT = _NP // 32
_AW = _RPT * _HALF
_K = 16
_STRIP = 3200
_NSTRIP = _E // _STRIP
_EPT32 = _E // 32
_CROWS = 768

_mesh = plsc.VectorSubcoreMesh(core_axis_name="c", subcore_axis_name="s")
_params = pltpu.CompilerParams(needs_layout_passes=False)


def _count_body(dst_hbm, out_hbm, dbuf, cnt, sem):
    c = lax.axis_index("c")
    s = lax.axis_index("s")
    w = c * 16 + s
    pltpu.async_copy(dst_hbm.at[pl.ds(w * _EPT32, _EPT32)],
                     dbuf.at[pl.ds(0, _EPT32)], sem).wait()
    zf = jnp.zeros((16,), jnp.float32)
    for r in range(_CROWS):
        cnt[pl.ds(r * 16, 16)] = zf
    ones = jnp.ones((16,), jnp.float32)
    iota = lax.iota(jnp.int32, 16)

    def body(i, _):
        off = i * 16
        v = dbuf[pl.ds(off, 16)]
        m = (off + iota) < _EPT32
        plsc.addupdate_scatter(cnt, [v], ones, mask=m)
        return 0

    lax.fori_loop(0, (_EPT32 + 15) // 16, body, 0)
    pltpu.sync_copy(cnt, out_hbm.at[w])


def _count(dst):
    f = pl.kernel(
        _count_body,
        out_type=jax.ShapeDtypeStruct((32, _CROWS * 16), jnp.float32),
        mesh=_mesh,
        compiler_params=_params,
        scratch_types=[
            pltpu.VMEM((_EPT32 + 8,), jnp.int32),
            pltpu.VMEM((_CROWS * 16,), jnp.float32),
            pltpu.SemaphoreType.DMA,
        ],
    )
    return f(dst)


def _agg_body(glo_hbm, ghi_hbm, src_hbm, dst_hbm, slo_hbm, shi_hbm,
              sbuf0, sbuf1, dbuf0, dbuf1, csrc, cdst,
              gbuf0, gbuf1, gbuf2, gbuf3, acc,
              seme0, seme1, semg0, semg1, semg2, semg3, sem):
    c = lax.axis_index("c")
    s = lax.axis_index("s")
    w = c * 16 + s
    lo = w * _RPT
    iota = lax.iota(jnp.int32, 16)
    zf = jnp.zeros((16,), jnp.float32)
    dummy_s = jnp.full((16,), _NP - 1, jnp.int32)
    dummy_d = jnp.zeros((16,), jnp.int32)
    cols = [j * 16 + iota for j in range(_HALF // 16)]
    lane = [jnp.full((16,), k, jnp.int32) for k in range(16)]
    sb = (sbuf0, sbuf1)
    db = (dbuf0, dbuf1)
    gb = (gbuf0, gbuf1, gbuf2, gbuf3)
    seme = (seme0, seme1)
    semg = (semg0, semg1, semg2, semg3)

    for half in range(2):
        g_hbm = glo_hbm if half == 0 else ghi_hbm
        o_hbm = slo_hbm if half == 0 else shi_hbm

        def zero(i, _):
            acc[pl.ds(i * 16, 16)] = zf
            return 0

        lax.fori_loop(0, _AW // 16, zero, 0)

        for b in range(2):
            pltpu.async_copy(src_hbm.at[pl.ds(b * _STRIP, _STRIP)],
                             sb[b], seme[b])
            pltpu.async_copy(dst_hbm.at[pl.ds(b * _STRIP, _STRIP)],
                             db[b], seme[b])

        def spair(sg, _s):
            for b in range(2):
                i = sg * 2 + b
                base_e = i * _STRIP
                sbuf, dbuf, sem_e = sb[b], db[b], seme[b]
                pltpu.make_async_copy(
                    src_hbm.at[pl.ds(base_e, _STRIP)], sbuf, sem_e).wait()
                pltpu.make_async_copy(
                    dst_hbm.at[pl.ds(base_e, _STRIP)], dbuf, sem_e).wait()

                def scan(ii, n):
                    dv = dbuf[pl.ds(ii * 16, 16)]
                    sv = sbuf[pl.ds(ii * 16, 16)]
                    m = (dv >= lo) & (dv < lo + _RPT)
                    plsc.store_compressed(csrc.at[pl.ds(n, 16)], sv, mask=m)
                    plsc.store_compressed(cdst.at[pl.ds(n, 16)], dv - lo,
                                          mask=m)
                    return n + jnp.sum(m.astype(jnp.int32))

                cnt = lax.fori_loop(0, _STRIP // 16, scan, jnp.int32(0))
                for k in range(_K // 16 + 1):
                    csrc[pl.ds(cnt + k * 16, 16)] = dummy_s
                    cdst[pl.ds(cnt + k * 16, 16)] = dummy_d

                @pl.when(i + 2 < _NSTRIP)
                def _():
                    nb = (i + 2) * _STRIP
                    pltpu.async_copy(src_hbm.at[pl.ds(nb, _STRIP)], sbuf,
                                     sem_e)
                    pltpu.async_copy(dst_hbm.at[pl.ds(nb, _STRIP)], dbuf,
                                     sem_e)

                nch = (cnt + _K - 1) // _K

                for pb in range(4):
                    @pl.when(pb < nch)
                    def _():
                        pltpu.async_copy(
                            g_hbm.at[csrc.at[pl.ds(pb * _K, _K)]],
                            gb[pb], semg[pb])

                def cquad(cg, _c):
                    for b2 in range(4):
                        cidx = cg * 4 + b2
                        gbuf, sem_g = gb[b2], semg[b2]

                        @pl.when(cidx < nch)
                        def _():
                            cb = cidx * _K
                            pltpu.make_async_copy(
                                g_hbm.at[csrc.at[pl.ds(cb, _K)]], gbuf,
                                sem_g).wait()

                            dv16 = cdst[pl.ds(cb, 16)]
                            rowb16 = dv16 * _HALF

                            def edge1(k, _e):
                                kv = jnp.zeros((16,), jnp.int32) + k
                                rowb = rowb16[kv]
                                for j in range(_HALF // 16):
                                    val = gbuf[k, j // 8,
                                               pl.ds((j % 8) * 16, 16)]
                                    plsc.addupdate_scatter(
                                        acc, [rowb + cols[j]], val)
                                return 0

                            lax.fori_loop(0, 16, edge1, 0)

                            @pl.when(cidx + 4 < nch)
                            def _():
                                nxt = (cidx + 4) * _K
                                pltpu.async_copy(
                                    g_hbm.at[csrc.at[pl.ds(nxt, _K)]],
                                    gbuf, sem_g)

                    return 0

                lax.fori_loop(0, (nch + 3) // 4, cquad, 0)
            return 0

        lax.fori_loop(0, _NSTRIP // 2, spair, 0)
        pltpu.sync_copy(acc, o_hbm.at[pl.ds(w * _AW, _AW)])


def _agg(glo, ghi, src, dst):
    f = pl.kernel(
        _agg_body,
        out_type=[
            jax.ShapeDtypeStruct((_NP * _HALF,), jnp.float32),
            jax.ShapeDtypeStruct((_NP * _HALF,), jnp.float32),
        ],
        mesh=_mesh,
        compiler_params=_params,
        scratch_types=[
            pltpu.VMEM((_STRIP,), jnp.int32),
            pltpu.VMEM((_STRIP,), jnp.int32),
            pltpu.VMEM((_STRIP,), jnp.int32),
            pltpu.VMEM((_STRIP,), jnp.int32),
            pltpu.VMEM((_STRIP + 2 * _K,), jnp.int32),
            pltpu.VMEM((_STRIP + 2 * _K,), jnp.int32),
            pltpu.VMEM((_K, _HALF // 128, 128), jnp.float32),
            pltpu.VMEM((_K, _HALF // 128, 128), jnp.float32),
            pltpu.VMEM((_K, _HALF // 128, 128), jnp.float32),
            pltpu.VMEM((_K, _HALF // 128, 128), jnp.float32),
            pltpu.VMEM((_AW,), jnp.float32),
            pltpu.SemaphoreType.DMA,
            pltpu.SemaphoreType.DMA,
            pltpu.SemaphoreType.DMA,
            pltpu.SemaphoreType.DMA,
            pltpu.SemaphoreType.DMA,
            pltpu.SemaphoreType.DMA,
            pltpu.SemaphoreType.DMA,
        ],
    )
    return f(glo.reshape(_NP, _HALF // 128, 128),
             ghi.reshape(_NP, _HALF // 128, 128), src, dst)


def _mm1_body(x_ref, w_ref, p_ref, glo_ref, ghi_ref, dis_ref):
    j = pl.program_id(0)
    deg = jnp.sum(p_ref[...], axis=1, keepdims=True) + 1.0
    rid = j * _BM + lax.broadcasted_iota(jnp.int32, (_BM, 1), 0)
    dis = jnp.where(rid < _N, lax.rsqrt(deg), 0.0)
    g = jnp.dot(x_ref[...], w_ref[...],
                preferred_element_type=jnp.float32) * dis
    glo_ref[...] = g[:, :_HALF]
    ghi_ref[...] = g[:, _HALF:]
    dis_ref[...] = dis


def _mm1(x_p, W1, parts):
    return pl.pallas_call(
        _mm1_body,
        grid=(_NBLK,),
        in_specs=[
            pl.BlockSpec((_BM, 256), lambda j: (j, 0)),
            pl.BlockSpec((256, _HID), lambda j: (0, 0)),
            pl.BlockSpec((_BM, 32), lambda j: (j, 0)),
        ],
        out_specs=[
            pl.BlockSpec((_BM, _HALF), lambda j: (j, 0)),
            pl.BlockSpec((_BM, _HALF), lambda j: (j, 0)),
            pl.BlockSpec((_BM, 1), lambda j: (j, 0)),
        ],
        out_shape=[
            jax.ShapeDtypeStruct((_NP, _HALF), jnp.float32),
            jax.ShapeDtypeStruct((_NP, _HALF), jnp.float32),
            jax.ShapeDtypeStruct((_NP, 1), jnp.float32),
        ],
    )(x_p, W1, parts)


def _mm2_body(slo_ref, shi_ref, glo_ref, ghi_ref, dis_ref, b_ref, w_ref,
              olo_ref, ohi_ref):
    dis = dis_ref[...]
    s = jnp.concatenate([slo_ref[...], shi_ref[...]], axis=1)
    g = jnp.concatenate([glo_ref[...], ghi_ref[...]], axis=1)
    h = jnp.maximum(dis * (s + g) + b_ref[...], 0.0)
    o = jnp.dot(h, w_ref[...], preferred_element_type=jnp.float32) * dis
    olo_ref[...] = o[:, :_HALF]
    ohi_ref[...] = o[:, _HALF:]


def _mm2(s1lo, s1hi, g1lo, g1hi, dis, b1, W2):
    return pl.pallas_call(
        _mm2_body,
        grid=(_NBLK,),
        in_specs=[
            pl.BlockSpec((_BM, _HALF), lambda j: (j, 0)),
            pl.BlockSpec((_BM, _HALF), lambda j: (j, 0)),
            pl.BlockSpec((_BM, _HALF), lambda j: (j, 0)),
            pl.BlockSpec((_BM, _HALF), lambda j: (j, 0)),
            pl.BlockSpec((_BM, 1), lambda j: (j, 0)),
            pl.BlockSpec((1, _HID), lambda j: (0, 0)),
            pl.BlockSpec((_HID, _HID), lambda j: (0, 0)),
        ],
        out_specs=[
            pl.BlockSpec((_BM, _HALF), lambda j: (j, 0)),
            pl.BlockSpec((_BM, _HALF), lambda j: (j, 0)),
        ],
        out_shape=[
            jax.ShapeDtypeStruct((_NP, _HALF), jnp.float32),
            jax.ShapeDtypeStruct((_NP, _HALF), jnp.float32),
        ],
    )(s1lo, s1hi, g1lo, g1hi, dis, b1, W2)


def _mm3_body(slo_ref, shi_ref, glo_ref, ghi_ref, dis_ref, b_ref, wc_ref,
              bc_ref, out_ref):
    dis = dis_ref[...]
    s = jnp.concatenate([slo_ref[...], shi_ref[...]], axis=1)
    g = jnp.concatenate([glo_ref[...], ghi_ref[...]], axis=1)
    h = jnp.maximum(dis * (s + g) + b_ref[...], 0.0)
    out_ref[...] = jnp.dot(h, wc_ref[...],
                           preferred_element_type=jnp.float32) + bc_ref[...]


def _mm3(s2lo, s2hi, g2lo, g2hi, dis, b2, Wcp, bcp):
    return pl.pallas_call(
        _mm3_body,
        grid=(_NBLK,),
        in_specs=[
            pl.BlockSpec((_BM, _HALF), lambda j: (j, 0)),
            pl.BlockSpec((_BM, _HALF), lambda j: (j, 0)),
            pl.BlockSpec((_BM, _HALF), lambda j: (j, 0)),
            pl.BlockSpec((_BM, _HALF), lambda j: (j, 0)),
            pl.BlockSpec((_BM, 1), lambda j: (j, 0)),
            pl.BlockSpec((1, _HID), lambda j: (0, 0)),
            pl.BlockSpec((_HID, _BM), lambda j: (0, 0)),
            pl.BlockSpec((1, _BM), lambda j: (0, 0)),
        ],
        out_specs=pl.BlockSpec((_BM, _BM), lambda j: (j, 0)),
        out_shape=jax.ShapeDtypeStruct((_NP, _BM), jnp.float32),
    )(s2lo, s2hi, g2lo, g2hi, dis, b2, Wcp, bcp)


def kernel(x, edge_index, W1, b1, W2, b2, Wc, bc):
    src = edge_index[0]
    dst = edge_index[1]
    x_p = jnp.pad(x, ((0, _NP - _N), (0, 0)))

    counts = _count(dst)
    parts = counts[:, :_NP].T

    g1lo, g1hi, dis = _mm1(x_p, W1, parts)
    s1lo, s1hi = _agg(g1lo, g1hi, src, dst)
    s1lo = s1lo.reshape(_NP, _HALF)
    s1hi = s1hi.reshape(_NP, _HALF)
    g2lo, g2hi = _mm2(s1lo, s1hi, g1lo, g1hi, dis, b1.reshape(1, -1), W2)
    s2lo, s2hi = _agg(g2lo, g2hi, src, dst)
    s2lo = s2lo.reshape(_NP, _HALF)
    s2hi = s2hi.reshape(_NP, _HALF)
    Wcp = jnp.pad(Wc, ((0, 0), (0, _BM - Wc.shape[1])))
    bcp = jnp.pad(bc, (0, _BM - bc.shape[0])).reshape(1, -1)
    out = _mm3(s2lo, s2hi, g2lo, g2hi, dis, b2.reshape(1, -1), Wcp, bcp)
    return out[:_N, :Wc.shape[1]]


# speedup vs baseline: 2.9318x; 1.0042x over previous
import jax
import jax.numpy as jnp
from jax import lax
from jax.experimental import pallas as pl
from jax.experimental.pallas import tpu as pltpu
from jax.experimental.pallas import tpu_sc as plsc

_N = 10000
_E = 160000
_NP = 10752
_HID = 512
_HALF = 256
_BM = 128
_NBLK = _NP // _BM
_RPT = _NP // 32
_AW = _RPT * _HALF
_K = 16
_STRIP = 3200
_NSTRIP = _E // _STRIP
_EPT32 = _E // 32
_CROWS = 768

_mesh = plsc.VectorSubcoreMesh(core_axis_name="c", subcore_axis_name="s")
_params = pltpu.CompilerParams(needs_layout_passes=False)


def _count_body(dst_hbm, out_hbm, dbuf, cnt, sem):
    c = lax.axis_index("c")
    s = lax.axis_index("s")
    w = c * 16 + s
    pltpu.async_copy(dst_hbm.at[pl.ds(w * _EPT32, _EPT32)],
                     dbuf.at[pl.ds(0, _EPT32)], sem).wait()
    zf = jnp.zeros((16,), jnp.float32)
    for r in range(_CROWS):
        cnt[pl.ds(r * 16, 16)] = zf
    ones = jnp.ones((16,), jnp.float32)
    iota = lax.iota(jnp.int32, 16)

    def body(i, _):
        off = i * 16
        v = dbuf[pl.ds(off, 16)]
        m = (off + iota) < _EPT32
        plsc.addupdate_scatter(cnt, [v], ones, mask=m)
        return 0

    lax.fori_loop(0, (_EPT32 + 15) // 16, body, 0)
    pltpu.sync_copy(cnt, out_hbm.at[w])


def _count(dst):
    f = pl.kernel(
        _count_body,
        out_type=jax.ShapeDtypeStruct((32, _CROWS * 16), jnp.float32),
        mesh=_mesh,
        compiler_params=_params,
        scratch_types=[
            pltpu.VMEM((_EPT32 + 8,), jnp.int32),
            pltpu.VMEM((_CROWS * 16,), jnp.float32),
            pltpu.SemaphoreType.DMA,
        ],
    )
    return f(dst)


def _agg_body(glo_hbm, ghi_hbm, src_hbm, dst_hbm, slo_hbm, shi_hbm,
              sbuf0, sbuf1, dbuf0, dbuf1, csrc, cdst,
              gbuf0, gbuf1, gbuf2, gbuf3, gbuf4, gbuf5, acc,
              seme0, seme1, semg0, semg1, semg2, semg3, semg4, semg5, sem):
    c = lax.axis_index("c")
    s = lax.axis_index("s")
    w = c * 16 + s
    lo = w * _RPT
    iota = lax.iota(jnp.int32, 16)
    zf = jnp.zeros((16,), jnp.float32)
    dummy_s = jnp.full((16,), _NP - 1, jnp.int32)
    dummy_d = jnp.zeros((16,), jnp.int32)
    cols = [j * 16 + iota for j in range(_HALF // 16)]
    lane = [jnp.full((16,), k, jnp.int32) for k in range(16)]
    sb = (sbuf0, sbuf1)
    db = (dbuf0, dbuf1)
    gb = (gbuf0, gbuf1, gbuf2, gbuf3, gbuf4, gbuf5)
    seme = (seme0, seme1)
    semg = (semg0, semg1, semg2, semg3, semg4, semg5)

    for half in range(2):
        g_hbm = glo_hbm if half == 0 else ghi_hbm
        o_hbm = slo_hbm if half == 0 else shi_hbm

        def zero(i, _):
            acc[pl.ds(i * 16, 16)] = zf
            return 0

        lax.fori_loop(0, _AW // 16, zero, 0)

        for b in range(2):
            pltpu.async_copy(src_hbm.at[pl.ds(b * _STRIP, _STRIP)],
                             sb[b], seme[b])
            pltpu.async_copy(dst_hbm.at[pl.ds(b * _STRIP, _STRIP)],
                             db[b], seme[b])

        def spair(sg, _s):
            for b in range(2):
                i = sg * 2 + b
                base_e = i * _STRIP
                sbuf, dbuf, sem_e = sb[b], db[b], seme[b]
                pltpu.make_async_copy(
                    src_hbm.at[pl.ds(base_e, _STRIP)], sbuf, sem_e).wait()
                pltpu.make_async_copy(
                    dst_hbm.at[pl.ds(base_e, _STRIP)], dbuf, sem_e).wait()

                def scan(ii, n):
                    dv = dbuf[pl.ds(ii * 16, 16)]
                    sv = sbuf[pl.ds(ii * 16, 16)]
                    m = (dv >= lo) & (dv < lo + _RPT)
                    plsc.store_compressed(csrc.at[pl.ds(n, 16)], sv, mask=m)
                    plsc.store_compressed(cdst.at[pl.ds(n, 16)], dv - lo,
                                          mask=m)
                    return n + jnp.sum(m.astype(jnp.int32))

                cnt = lax.fori_loop(0, _STRIP // 16, scan, jnp.int32(0))
                for k in range(_K // 16 + 1):
                    csrc[pl.ds(cnt + k * 16, 16)] = dummy_s
                    cdst[pl.ds(cnt + k * 16, 16)] = dummy_d

                @pl.when(i + 2 < _NSTRIP)
                def _():
                    nb = (i + 2) * _STRIP
                    pltpu.async_copy(src_hbm.at[pl.ds(nb, _STRIP)], sbuf,
                                     sem_e)
                    pltpu.async_copy(dst_hbm.at[pl.ds(nb, _STRIP)], dbuf,
                                     sem_e)

                nch = (cnt + _K - 1) // _K

                for pb in range(6):
                    @pl.when(pb < nch)
                    def _():
                        pltpu.async_copy(
                            g_hbm.at[csrc.at[pl.ds(pb * _K, _K)]],
                            gb[pb], semg[pb])

                def cquad(cg, _c):
                    for b2 in range(6):
                        cidx = cg * 6 + b2
                        gbuf, sem_g = gb[b2], semg[b2]

                        @pl.when(cidx < nch)
                        def _():
                            cb = cidx * _K
                            pltpu.make_async_copy(
                                g_hbm.at[csrc.at[pl.ds(cb, _K)]], gbuf,
                                sem_g).wait()

                            dv16 = cdst[pl.ds(cb, 16)]
                            rowb16 = dv16 * _HALF

                            def edge1(k, _e):
                                kv = jnp.zeros((16,), jnp.int32) + k
                                rowb = rowb16[kv]
                                for j in range(_HALF // 16):
                                    val = gbuf[k, j // 8,
                                               pl.ds((j % 8) * 16, 16)]
                                    plsc.addupdate_scatter(
                                        acc, [rowb + cols[j]], val)
                                return 0

                            lax.fori_loop(0, 16, edge1, 0)

                            @pl.when(cidx + 6 < nch)
                            def _():
                                nxt = (cidx + 6) * _K
                                pltpu.async_copy(
                                    g_hbm.at[csrc.at[pl.ds(nxt, _K)]],
                                    gbuf, sem_g)

                    return 0

                lax.fori_loop(0, (nch + 5) // 6, cquad, 0)
            return 0

        lax.fori_loop(0, _NSTRIP // 2, spair, 0)
        pltpu.sync_copy(acc, o_hbm.at[pl.ds(w * _AW, _AW)])


def _agg(glo, ghi, src, dst):
    f = pl.kernel(
        _agg_body,
        out_type=[
            jax.ShapeDtypeStruct((_NP * _HALF,), jnp.float32),
            jax.ShapeDtypeStruct((_NP * _HALF,), jnp.float32),
        ],
        mesh=_mesh,
        compiler_params=_params,
        scratch_types=[
            pltpu.VMEM((_STRIP,), jnp.int32),
            pltpu.VMEM((_STRIP,), jnp.int32),
            pltpu.VMEM((_STRIP,), jnp.int32),
            pltpu.VMEM((_STRIP,), jnp.int32),
            pltpu.VMEM((_STRIP + 2 * _K,), jnp.int32),
            pltpu.VMEM((_STRIP + 2 * _K,), jnp.int32),
            pltpu.VMEM((_K, _HALF // 128, 128), jnp.float32),
            pltpu.VMEM((_K, _HALF // 128, 128), jnp.float32),
            pltpu.VMEM((_K, _HALF // 128, 128), jnp.float32),
            pltpu.VMEM((_K, _HALF // 128, 128), jnp.float32),
            pltpu.VMEM((_K, _HALF // 128, 128), jnp.float32),
            pltpu.VMEM((_K, _HALF // 128, 128), jnp.float32),
            pltpu.VMEM((_AW,), jnp.float32),
            pltpu.SemaphoreType.DMA,
            pltpu.SemaphoreType.DMA,
            pltpu.SemaphoreType.DMA,
            pltpu.SemaphoreType.DMA,
            pltpu.SemaphoreType.DMA,
            pltpu.SemaphoreType.DMA,
            pltpu.SemaphoreType.DMA,
            pltpu.SemaphoreType.DMA,
            pltpu.SemaphoreType.DMA,
        ],
    )
    return f(glo.reshape(_NP, _HALF // 128, 128),
             ghi.reshape(_NP, _HALF // 128, 128), src, dst)


def _mm1_body(x_ref, w_ref, p_ref, glo_ref, ghi_ref, dis_ref):
    j = pl.program_id(0)
    deg = jnp.sum(p_ref[...], axis=1, keepdims=True) + 1.0
    rid = j * _BM + lax.broadcasted_iota(jnp.int32, (_BM, 1), 0)
    dis = jnp.where(rid < _N, lax.rsqrt(deg), 0.0)
    g = jnp.dot(x_ref[...], w_ref[...],
                preferred_element_type=jnp.float32) * dis
    glo_ref[...] = g[:, :_HALF]
    ghi_ref[...] = g[:, _HALF:]
    dis_ref[...] = dis


def _mm1(x_p, W1, parts):
    return pl.pallas_call(
        _mm1_body,
        grid=(_NBLK,),
        in_specs=[
            pl.BlockSpec((_BM, 256), lambda j: (j, 0)),
            pl.BlockSpec((256, _HID), lambda j: (0, 0)),
            pl.BlockSpec((_BM, 32), lambda j: (j, 0)),
        ],
        out_specs=[
            pl.BlockSpec((_BM, _HALF), lambda j: (j, 0)),
            pl.BlockSpec((_BM, _HALF), lambda j: (j, 0)),
            pl.BlockSpec((_BM, 1), lambda j: (j, 0)),
        ],
        out_shape=[
            jax.ShapeDtypeStruct((_NP, _HALF), jnp.float32),
            jax.ShapeDtypeStruct((_NP, _HALF), jnp.float32),
            jax.ShapeDtypeStruct((_NP, 1), jnp.float32),
        ],
    )(x_p, W1, parts)


def _mm2_body(slo_ref, shi_ref, glo_ref, ghi_ref, dis_ref, b_ref, w_ref,
              olo_ref, ohi_ref):
    dis = dis_ref[...]
    s = jnp.concatenate([slo_ref[...], shi_ref[...]], axis=1)
    g = jnp.concatenate([glo_ref[...], ghi_ref[...]], axis=1)
    h = jnp.maximum(dis * (s + g) + b_ref[...], 0.0)
    o = jnp.dot(h, w_ref[...], preferred_element_type=jnp.float32) * dis
    olo_ref[...] = o[:, :_HALF]
    ohi_ref[...] = o[:, _HALF:]


def _mm2(s1lo, s1hi, g1lo, g1hi, dis, b1, W2):
    return pl.pallas_call(
        _mm2_body,
        grid=(_NBLK,),
        in_specs=[
            pl.BlockSpec((_BM, _HALF), lambda j: (j, 0)),
            pl.BlockSpec((_BM, _HALF), lambda j: (j, 0)),
            pl.BlockSpec((_BM, _HALF), lambda j: (j, 0)),
            pl.BlockSpec((_BM, _HALF), lambda j: (j, 0)),
            pl.BlockSpec((_BM, 1), lambda j: (j, 0)),
            pl.BlockSpec((1, _HID), lambda j: (0, 0)),
            pl.BlockSpec((_HID, _HID), lambda j: (0, 0)),
        ],
        out_specs=[
            pl.BlockSpec((_BM, _HALF), lambda j: (j, 0)),
            pl.BlockSpec((_BM, _HALF), lambda j: (j, 0)),
        ],
        out_shape=[
            jax.ShapeDtypeStruct((_NP, _HALF), jnp.float32),
            jax.ShapeDtypeStruct((_NP, _HALF), jnp.float32),
        ],
    )(s1lo, s1hi, g1lo, g1hi, dis, b1, W2)


def _mm3_body(slo_ref, shi_ref, glo_ref, ghi_ref, dis_ref, b_ref, wc_ref,
              bc_ref, out_ref):
    dis = dis_ref[...]
    s = jnp.concatenate([slo_ref[...], shi_ref[...]], axis=1)
    g = jnp.concatenate([glo_ref[...], ghi_ref[...]], axis=1)
    h = jnp.maximum(dis * (s + g) + b_ref[...], 0.0)
    out_ref[...] = jnp.dot(h, wc_ref[...],
                           preferred_element_type=jnp.float32) + bc_ref[...]


def _mm3(s2lo, s2hi, g2lo, g2hi, dis, b2, Wcp, bcp):
    return pl.pallas_call(
        _mm3_body,
        grid=(_NBLK,),
        in_specs=[
            pl.BlockSpec((_BM, _HALF), lambda j: (j, 0)),
            pl.BlockSpec((_BM, _HALF), lambda j: (j, 0)),
            pl.BlockSpec((_BM, _HALF), lambda j: (j, 0)),
            pl.BlockSpec((_BM, _HALF), lambda j: (j, 0)),
            pl.BlockSpec((_BM, 1), lambda j: (j, 0)),
            pl.BlockSpec((1, _HID), lambda j: (0, 0)),
            pl.BlockSpec((_HID, _BM), lambda j: (0, 0)),
            pl.BlockSpec((1, _BM), lambda j: (0, 0)),
        ],
        out_specs=pl.BlockSpec((_BM, _BM), lambda j: (j, 0)),
        out_shape=jax.ShapeDtypeStruct((_NP, _BM), jnp.float32),
    )(s2lo, s2hi, g2lo, g2hi, dis, b2, Wcp, bcp)


def kernel(x, edge_index, W1, b1, W2, b2, Wc, bc):
    src = edge_index[0]
    dst = edge_index[1]
    x_p = jnp.pad(x, ((0, _NP - _N), (0, 0)))

    counts = _count(dst)
    parts = counts[:, :_NP].T

    g1lo, g1hi, dis = _mm1(x_p, W1, parts)
    s1lo, s1hi = _agg(g1lo, g1hi, src, dst)
    s1lo = s1lo.reshape(_NP, _HALF)
    s1hi = s1hi.reshape(_NP, _HALF)
    g2lo, g2hi = _mm2(s1lo, s1hi, g1lo, g1hi, dis, b1.reshape(1, -1), W2)
    s2lo, s2hi = _agg(g2lo, g2hi, src, dst)
    s2lo = s2lo.reshape(_NP, _HALF)
    s2hi = s2hi.reshape(_NP, _HALF)
    Wcp = jnp.pad(Wc, ((0, 0), (0, _BM - Wc.shape[1])))
    bcp = jnp.pad(bc, (0, _BM - bc.shape[0])).reshape(1, -1)
    out = _mm3(s2lo, s2hi, g2lo, g2hi, dis, b2.reshape(1, -1), Wcp, bcp)
    return out[:_N, :Wc.shape[1]]
